# baseline (speedup 1.0000x reference)
import jax
import jax.numpy as jnp
from jax import lax
from jax.experimental import pallas as pl
from jax.experimental.pallas import tpu as pltpu
from jax.experimental.pallas import tpu_sc as plsc

N = 1_000_000
NW = 32
L = 16
BLK = 128
CHUNK = 2_048
CBLK = CHUNK // BLK
NBLK = 7_813
NP = 1_000_448
PBLK = NP // BLK
MAIN_CHUNKS = 488
TAIL_J0 = MAIN_CHUNKS * CBLK
TAIL_BLOCKS = 5
ROT_MAIN_BLOCKS = 7_812
ROT_TAIL_PTS = N - ROT_MAIN_BLOCKS * BLK


def _rsqrt_nr(a):
    i = lax.bitcast_convert_type(a, jnp.int32)
    i = jnp.int32(0x5F3759DF) - (i >> 1)
    y = lax.bitcast_convert_type(i, jnp.float32)
    ah = 0.5 * a
    for _ in range(3):
        y = y * (1.5 - ah * y * y)
    return y


def _compute_group(jj, gg, ls_v, rot_v, dr_v, di_v, out_v):
    p0 = jj * BLK + gg * L
    sl = pl.ds(p0, L)
    r0 = jj * (4 * BLK) + gg * L

    sx = jnp.exp(ls_v[0, sl])
    sy = jnp.exp(ls_v[1, sl])
    sz = jnp.exp(ls_v[2, sl])
    qw = rot_v[pl.ds(r0, L)]
    qx = rot_v[pl.ds(r0 + BLK, L)]
    qy = rot_v[pl.ds(r0 + 2 * BLK, L)]
    qz = rot_v[pl.ds(r0 + 3 * BLK, L)]
    dr = dr_v[sl]
    di = di_v[sl]

    n2 = qw * qw + qx * qx + qy * qy + qz * qz
    norm = n2 * _rsqrt_nr(n2)
    inv = 1.0 / (norm + 1e-8)
    w, x, y, z = qw * inv, qx * inv, qy * inv, qz * inv

    xx, yy, zz = x * x, y * y, z * z
    xy, xz, yz = x * y, x * z, y * z
    wx, wy, wz = w * x, w * y, w * z
    r00 = 1.0 - 2.0 * (yy + zz)
    r01 = 2.0 * (xy - wz)
    r02 = 2.0 * (xz + wy)
    r10 = 2.0 * (xy + wz)
    r11 = 1.0 - 2.0 * (xx + zz)
    r12 = 2.0 * (yz - wx)
    r20 = 2.0 * (xz - wy)
    r21 = 2.0 * (yz + wx)
    r22 = 1.0 - 2.0 * (xx + yy)

    s2x, s2y, s2z = sx * sx, sy * sy, sz * sz
    a00, a01, a02 = r00 * s2x, r01 * s2y, r02 * s2z
    a10, a11, a12 = r10 * s2x, r11 * s2y, r12 * s2z
    a20, a21, a22 = r20 * s2x, r21 * s2y, r22 * s2z
    s00 = a00 * r00 + a01 * r01 + a02 * r02
    s01 = a00 * r10 + a01 * r11 + a02 * r12
    s02 = a00 * r20 + a01 * r21 + a02 * r22
    s11 = a10 * r10 + a11 * r11 + a12 * r12
    s12 = a10 * r20 + a11 * r21 + a12 * r22
    s22 = a20 * r20 + a21 * r21 + a22 * r22

    t = dr * dr + di * di + 1e-12
    dmag = t * _rsqrt_nr(t)

    vals = ((0, 3, s00), (0, 4, s01), (0, 5, s02),
            (0, 6, s01), (0, 7, s11),
            (1, 0, s12), (1, 1, s02), (1, 2, s12), (1, 3, s22),
            (1, 4, sx), (1, 5, sy), (1, 6, sz), (1, 7, dmag))
    l0 = gg * L
    for h, r, v in vals:
        out_v[h, jj, r, pl.ds(l0, L)] = v


def _make_chunk_fns(ls_hbm, rotm_hbm, rott_hbm, dr_hbm, di_hbm,
                    pos_hbm, out_hbm,
                    ls_v, rot_v, dr_v, di_v, out_v, sem_in, sem_out):
    def in_descs(j0, b, nblk, npts_d):
        npts = nblk * BLK
        descs = [
            pltpu.make_async_copy(
                ls_hbm.at[pl.ds(c * NP + j0 * BLK, npts)],
                ls_v[b].at[c, pl.ds(0, npts)], sem_in[b])
            for c in range(3)
        ]
        descs.append(pltpu.make_async_copy(
            dr_hbm.at[pl.ds(j0 * BLK, npts_d)],
            dr_v[b].at[pl.ds(0, npts_d)], sem_in[b]))
        descs.append(pltpu.make_async_copy(
            di_hbm.at[pl.ds(j0 * BLK, npts_d)],
            di_v[b].at[pl.ds(0, npts_d)], sem_in[b]))
        return descs

    def rot_main_desc(j0, b, nblk):
        return pltpu.make_async_copy(
            rotm_hbm.at[pl.ds(j0 * 4 * BLK, nblk * 4 * BLK)],
            rot_v[b].at[pl.ds(0, nblk * 4 * BLK)], sem_in[b])

    def rot_tail_desc(b):
        return pltpu.make_async_copy(
            rott_hbm, rot_v[b].at[pl.ds(4 * 4 * BLK, 4 * BLK)], sem_in[b])

    def out_descs(j0, b, nblk):
        descs = [
            pltpu.make_async_copy(out_v[b].at[h, pl.ds(0, nblk)],
                                  out_hbm.at[h, pl.ds(j0, nblk)], sem_out[b])
            for h in (0, 1)
        ]
        descs += [
            pltpu.make_async_copy(pos_hbm.at[c, pl.ds(j0, nblk)],
                                  out_hbm.at[0, pl.ds(j0, nblk), c],
                                  sem_out[b])
            for c in range(3)
        ]
        return descs

    def compute(b, nblk):
        def blk_body(jj, c1):
            for gg in range(BLK // L):
                _compute_group(jj, gg, ls_v[b], rot_v[b],
                               dr_v[b], di_v[b], out_v[b])
            return c1
        lax.fori_loop(0, nblk, blk_body, 0)

    return in_descs, rot_main_desc, rot_tail_desc, out_descs, compute


def _sc_kernel(ls_hbm, rotm_hbm, rott_hbm, dr_hbm, di_hbm, pos_hbm, out_hbm,
               ls_v, rot_v, dr_v, di_v, out_v, sem_in, sem_out):
    wid = lax.axis_index("s") * 2 + lax.axis_index("c")
    nslots = 15 + (wid < 8).astype(jnp.int32)

    (in_descs, rot_main_desc, rot_tail_desc, out_descs,
     compute) = _make_chunk_fns(
        ls_hbm, rotm_hbm, rott_hbm, dr_hbm, di_hbm, pos_hbm, out_hbm,
        ls_v, rot_v, dr_v, di_v, out_v, sem_in, sem_out)

    def chunk_j0(slot):
        return (wid + NW * slot) * CBLK

    def issue_in(slot, b):
        for d in in_descs(chunk_j0(slot), b, CBLK, CHUNK):
            d.start()
        rot_main_desc(chunk_j0(slot), b, CBLK).start()

    def wait_in(slot, b):
        for d in in_descs(chunk_j0(slot), b, CBLK, CHUNK):
            d.wait()
        rot_main_desc(chunk_j0(slot), b, CBLK).wait()

    issue_in(0, 0)

    def body(t, carry):
        for b in (0, 1):
            g = 2 * t + b

            @pl.when(g < nslots)
            def _():
                wait_in(g, b)

                @pl.when(g + 1 < nslots)
                def _():
                    issue_in(g + 1, 1 - b)

                @pl.when(g >= 2)
                def _():
                    for d in out_descs(chunk_j0(g - 2), b, CBLK):
                        d.wait()

                compute(b, CBLK)

                for d in out_descs(chunk_j0(g), b, CBLK):
                    d.start()
        return carry

    lax.fori_loop(0, 8, body, 0)

    slot_b1 = jnp.where(nslots == 16, 15, 13)
    for d in out_descs(chunk_j0(14), 0, CBLK):
        d.wait()
    for d in out_descs(chunk_j0(slot_b1), 1, CBLK):
        d.wait()

    @pl.when(wid == NW - 1)
    def _():
        tail_in = in_descs(TAIL_J0, 0, TAIL_BLOCKS, N - TAIL_J0 * BLK)
        tail_in.append(pltpu.make_async_copy(
            rotm_hbm.at[pl.ds(TAIL_J0 * 4 * BLK, 4 * 4 * BLK)],
            rot_v[0].at[pl.ds(0, 4 * 4 * BLK)], sem_in[0]))
        tail_in.append(rot_tail_desc(0))
        for d in tail_in:
            d.start()
        for d in tail_in:
            d.wait()
        compute(0, TAIL_BLOCKS)
        for d in out_descs(TAIL_J0, 0, TAIL_BLOCKS):
            d.start()
        for d in out_descs(TAIL_J0, 0, TAIL_BLOCKS):
            d.wait()


def kernel(positions, log_scales, rotations, density_real, density_imag):
    mesh = plsc.VectorSubcoreMesh(core_axis_name="c", subcore_axis_name="s")
    f = pl.kernel(
        _sc_kernel,
        out_type=jax.ShapeDtypeStruct((2, NBLK, 8, BLK), jnp.float32),
        mesh=mesh,
        compiler_params=pltpu.CompilerParams(
            needs_layout_passes=False, use_tc_tiling_on_sc=False),
        scratch_types=[
            [pltpu.VMEM((3, CHUNK), jnp.float32) for _ in range(2)],
            [pltpu.VMEM((CBLK * 4 * BLK,), jnp.float32) for _ in range(2)],
            [pltpu.VMEM((CHUNK,), jnp.float32) for _ in range(2)],
            [pltpu.VMEM((CHUNK,), jnp.float32) for _ in range(2)],
            [pltpu.VMEM((2, CBLK, 8, BLK), jnp.float32) for _ in range(2)],
            [pltpu.SemaphoreType.DMA for _ in range(2)],
            [pltpu.SemaphoreType.DMA for _ in range(2)],
        ],
    )

    def planar(a, ncols):
        ap = jnp.pad(a, ((0, NP - N), (0, 0)))
        return ap.T.reshape(ncols * PBLK * BLK)

    rot_main = (rotations[:ROT_MAIN_BLOCKS * BLK]
                .reshape(ROT_MAIN_BLOCKS, BLK, 4)
                .transpose(0, 2, 1)
                .reshape(ROT_MAIN_BLOCKS * 4 * BLK))
    rot_tail = jnp.pad(rotations[ROT_MAIN_BLOCKS * BLK:].T,
                       ((0, 0), (0, BLK - ROT_TAIL_PTS))).reshape(4 * BLK)

    pos_planar = planar(positions, 3).reshape(3, PBLK, BLK)

    out4 = f(planar(log_scales, 3), rot_main, rot_tail,
             density_real, density_imag, pos_planar)
    out = out4.transpose(1, 3, 0, 2).reshape(NBLK * BLK, 16)
    return out[:N]


# speedup vs baseline: 2.5897x; 2.5897x over previous
import jax
import jax.numpy as jnp
from jax import lax
from jax.experimental import pallas as pl
from jax.experimental.pallas import tpu as pltpu
from jax.experimental.pallas import tpu_sc as plsc

N = 1_000_000
NW = 32
L = 16
BLK = 128
CHUNK = 2_048
NBLK = 7_813
NP = 1_000_448
PBLK = NP // BLK
MAIN_CHUNKS = 488
TAIL_START = MAIN_CHUNKS * CHUNK
TAIL_BLOCKS = 5


def _rsqrt_nr(a):
    i = lax.bitcast_convert_type(a, jnp.int32)
    i = jnp.int32(0x5F3759DF) - (i >> 1)
    y = lax.bitcast_convert_type(i, jnp.float32)
    ah = 0.5 * a
    for _ in range(3):
        y = y * (1.5 - ah * y * y)
    return y


def _compute_group(jj, gg, pos_v, ls_v, rot_v, dr_v, di_v, out_v):
    p0 = jj * BLK + gg * L
    sl = pl.ds(p0, L)

    b3 = jj * (3 * BLK) + gg * L
    b4 = jj * (4 * BLK) + gg * L
    px = pos_v[pl.ds(b3, L)]
    py = pos_v[pl.ds(b3 + BLK, L)]
    pz = pos_v[pl.ds(b3 + 2 * BLK, L)]
    sx = jnp.exp(ls_v[pl.ds(b3, L)])
    sy = jnp.exp(ls_v[pl.ds(b3 + BLK, L)])
    sz = jnp.exp(ls_v[pl.ds(b3 + 2 * BLK, L)])
    qw = rot_v[pl.ds(b4, L)]
    qx = rot_v[pl.ds(b4 + BLK, L)]
    qy = rot_v[pl.ds(b4 + 2 * BLK, L)]
    qz = rot_v[pl.ds(b4 + 3 * BLK, L)]
    dr = dr_v[sl]
    di = di_v[sl]

    n2 = qw * qw + qx * qx + qy * qy + qz * qz
    norm = n2 * _rsqrt_nr(n2)
    inv = 1.0 / (norm + 1e-8)
    w, x, y, z = qw * inv, qx * inv, qy * inv, qz * inv

    xx, yy, zz = x * x, y * y, z * z
    xy, xz, yz = x * y, x * z, y * z
    wx, wy, wz = w * x, w * y, w * z
    r00 = 1.0 - 2.0 * (yy + zz)
    r01 = 2.0 * (xy - wz)
    r02 = 2.0 * (xz + wy)
    r10 = 2.0 * (xy + wz)
    r11 = 1.0 - 2.0 * (xx + zz)
    r12 = 2.0 * (yz - wx)
    r20 = 2.0 * (xz - wy)
    r21 = 2.0 * (yz + wx)
    r22 = 1.0 - 2.0 * (xx + yy)

    s2x, s2y, s2z = sx * sx, sy * sy, sz * sz
    a00, a01, a02 = r00 * s2x, r01 * s2y, r02 * s2z
    a10, a11, a12 = r10 * s2x, r11 * s2y, r12 * s2z
    a20, a21, a22 = r20 * s2x, r21 * s2y, r22 * s2z
    s00 = a00 * r00 + a01 * r01 + a02 * r02
    s01 = a00 * r10 + a01 * r11 + a02 * r12
    s02 = a00 * r20 + a01 * r21 + a02 * r22
    s11 = a10 * r10 + a11 * r11 + a12 * r12
    s12 = a10 * r20 + a11 * r21 + a12 * r22
    s22 = a20 * r20 + a21 * r21 + a22 * r22

    t = dr * dr + di * di + 1e-12
    dmag = t * _rsqrt_nr(t)

    vals = (px, py, pz,
            s00, s01, s02, s01, s11, s12, s02, s12, s22,
            sx, sy, sz, dmag)
    l0 = gg * L
    for c in range(16):
        out_v[c // 8, jj, c % 8, pl.ds(l0, L)] = vals[c]


def _make_chunk_fns(pos_hbm, ls_hbm, rot_hbm, dr_hbm, di_hbm, out_hbm,
                    pos_v, ls_v, rot_v, dr_v, di_v, out_v,
                    sem_in, sem_out):
    def in_descs(j0, b, nblk, npts_d):
        npts = nblk * BLK
        descs = []
        for src, dst, nc in ((pos_hbm, pos_v[b], 3), (ls_hbm, ls_v[b], 3),
                             (rot_hbm, rot_v[b], 4)):
            descs.append(pltpu.make_async_copy(
                src.at[pl.ds(j0 * nc * BLK, npts * nc)],
                dst.at[pl.ds(0, npts * nc)], sem_in[b]))
        descs.append(pltpu.make_async_copy(
            dr_hbm.at[pl.ds(j0 * BLK, npts_d)],
            dr_v[b].at[pl.ds(0, npts_d)], sem_in[b]))
        descs.append(pltpu.make_async_copy(
            di_hbm.at[pl.ds(j0 * BLK, npts_d)],
            di_v[b].at[pl.ds(0, npts_d)], sem_in[b]))
        return descs

    def out_descs(j0, b, nblk):
        return [
            pltpu.make_async_copy(out_v[b].at[h, pl.ds(0, nblk)],
                                  out_hbm.at[h, pl.ds(j0, nblk)], sem_out[b])
            for h in (0, 1)
        ]

    def compute(b, nblk):
        def blk_body(jj, c1):
            for gg in range(BLK // L):
                _compute_group(jj, gg, pos_v[b], ls_v[b], rot_v[b],
                               dr_v[b], di_v[b], out_v[b])
            return c1
        lax.fori_loop(0, nblk, blk_body, 0)

    return in_descs, out_descs, compute


def _sc_kernel(pos_hbm, ls_hbm, rot_hbm, dr_hbm, di_hbm, out_hbm,
               pos_v, ls_v, rot_v, dr_v, di_v, out_v,
               sem_in, sem_out):
    wid = lax.axis_index("s") * 2 + lax.axis_index("c")
    nslots = 15 + (wid < 8).astype(jnp.int32)

    in_descs, out_descs, compute = _make_chunk_fns(
        pos_hbm, ls_hbm, rot_hbm, dr_hbm, di_hbm, out_hbm,
        pos_v, ls_v, rot_v, dr_v, di_v, out_v, sem_in, sem_out)

    def chunk_j0(slot):
        return (wid + NW * slot) * (CHUNK // BLK)

    def issue_in(slot, b):
        for d in in_descs(chunk_j0(slot), b, CHUNK // BLK, CHUNK):
            d.start()

    def wait_in(slot, b):
        for d in in_descs(chunk_j0(slot), b, CHUNK // BLK, CHUNK):
            d.wait()

    issue_in(0, 0)

    def body(t, carry):
        for b in (0, 1):
            g = 2 * t + b

            @pl.when(g < nslots)
            def _():
                wait_in(g, b)

                @pl.when(g + 1 < nslots)
                def _():
                    issue_in(g + 1, 1 - b)

                @pl.when(g >= 2)
                def _():
                    for d in out_descs(chunk_j0(g - 2), b, CHUNK // BLK):
                        d.wait()

                compute(b, CHUNK // BLK)

                for d in out_descs(chunk_j0(g), b, CHUNK // BLK):
                    d.start()
        return carry

    lax.fori_loop(0, 8, body, 0)

    slot_b0 = jnp.where(nslots == 16, 14, 14)
    slot_b1 = jnp.where(nslots == 16, 15, 13)
    for d in out_descs(chunk_j0(slot_b0), 0, CHUNK // BLK):
        d.wait()
    for d in out_descs(chunk_j0(slot_b1), 1, CHUNK // BLK):
        d.wait()

    @pl.when(wid == NW - 1)
    def _():
        j0 = TAIL_START // BLK
        for d in in_descs(j0, 0, TAIL_BLOCKS, N - TAIL_START):
            d.start()
        for d in in_descs(j0, 0, TAIL_BLOCKS, N - TAIL_START):
            d.wait()
        compute(0, TAIL_BLOCKS)
        for d in out_descs(j0, 0, TAIL_BLOCKS):
            d.start()
        for d in out_descs(j0, 0, TAIL_BLOCKS):
            d.wait()


def kernel(positions, log_scales, rotations, density_real, density_imag):
    mesh = plsc.VectorSubcoreMesh(core_axis_name="c", subcore_axis_name="s")
    f = pl.kernel(
        _sc_kernel,
        out_type=jax.ShapeDtypeStruct((2, NBLK, 8, BLK), jnp.float32),
        mesh=mesh,
        compiler_params=pltpu.CompilerParams(
            needs_layout_passes=False, use_tc_tiling_on_sc=False),
        scratch_types=[
            [pltpu.VMEM((3 * CHUNK,), jnp.float32) for _ in range(2)],
            [pltpu.VMEM((3 * CHUNK,), jnp.float32) for _ in range(2)],
            [pltpu.VMEM((4 * CHUNK,), jnp.float32) for _ in range(2)],
            [pltpu.VMEM((CHUNK,), jnp.float32) for _ in range(2)],
            [pltpu.VMEM((CHUNK,), jnp.float32) for _ in range(2)],
            [pltpu.VMEM((2, CHUNK // BLK, 8, BLK), jnp.float32)
             for _ in range(2)],
            [pltpu.SemaphoreType.DMA for _ in range(2)],
            [pltpu.SemaphoreType.DMA for _ in range(2)],
        ],
    )

    def planar(a, ncols):
        ap = jnp.pad(a, ((0, NP - N), (0, 0)))
        return ap.reshape(PBLK, BLK, ncols).transpose(0, 2, 1).reshape(-1)

    out4 = f(planar(positions, 3), planar(log_scales, 3),
             planar(rotations, 4), density_real, density_imag)
    out = out4.transpose(1, 3, 0, 2).reshape(NBLK * BLK, 16)
    return out[:N]
